# baseline (speedup 1.0000x reference)
import functools

import jax
import jax.numpy as jnp
from jax import lax
from jax.experimental import pallas as pl
from jax.experimental.pallas import tpu as pltpu
from jax.experimental.pallas import tpu_sc as plsc

NN = 10000
EE = 320000
DD = 128
DW = DD + 16
NG = 16

NC = 2
NS = 16
CH = 56
CPT = 180
EPAD = NC * NS * CPT * CH
EHALF = EPAD // 2
NPAD = NN + 112
RPT = NPAD // NS

_PREC = lax.Precision.HIGHEST


def _dot16(a, b):
    return jnp.dot(a.astype(jnp.bfloat16), b.astype(jnp.bfloat16),
                   preferred_element_type=jnp.float32)


def _mm2(x, wa, wb, interpret=False):
    n = x.shape[0]
    blk = 1000
    grid = n // blk

    def body(x_ref, wa_ref, wb_ref, oa_ref, ob_ref):
        xb = x_ref[...]
        ha = _dot16(xb, wa_ref[...])
        oa_ref[...] = jnp.concatenate(
            [ha, jnp.ones((blk, 16), jnp.float32)], axis=1)
        ob_ref[...] = _dot16(xb, wb_ref[...])

    return pl.pallas_call(
        body,
        grid=(grid,),
        in_specs=[
            pl.BlockSpec((blk, DD), lambda i: (i, 0)),
            pl.BlockSpec((DD, DD), lambda i: (0, 0)),
            pl.BlockSpec((DD, DD), lambda i: (0, 0)),
        ],
        out_specs=[
            pl.BlockSpec((blk, DW), lambda i: (i, 0)),
            pl.BlockSpec((blk, DD), lambda i: (i, 0)),
        ],
        out_shape=[
            jax.ShapeDtypeStruct((n, DW), jnp.float32),
            jax.ShapeDtypeStruct((n, DD), jnp.float32),
        ],
        interpret=interpret,
    )(x, wa, wb)


def _edge_pass(hs, ht, srcp, dstp, avec, interpret=False):
    mesh = plsc.VectorSubcoreMesh(core_axis_name="c", subcore_axis_name="s",
                                  num_cores=NC, num_subcores=NS)

    @functools.partial(
        pl.kernel,
        out_type=jax.ShapeDtypeStruct((NC, NPAD, DW), jnp.float32),
        mesh=mesh,
        scratch_types=[
            pltpu.VMEM_SHARED((NPAD, DW), jnp.float32),
            [pltpu.VMEM((CH,), jnp.int32) for _ in range(3)],
            [pltpu.VMEM((CH,), jnp.int32) for _ in range(3)],
            [pltpu.VMEM((CH, DW), jnp.float32) for _ in range(2)],
            [pltpu.VMEM((CH, DD), jnp.float32) for _ in range(2)],
            pltpu.VMEM((DD,), jnp.float32),
            [pltpu.SemaphoreType.DMA for _ in range(3)],
            [pltpu.SemaphoreType.DMA for _ in range(2)],
            [pltpu.SemaphoreType.DMA for _ in range(2)],
        ],
        compiler_params=pltpu.CompilerParams(needs_layout_passes=False,
                                             use_tc_tiling_on_sc=False),
        interpret=interpret,
    )
    def edge_kernel(hs_h, ht_h, src_h, dst_h, a_h, out_h,
                    nacc, sidx, didx, sbuf, tbuf, abuf, semi, semg, semsc):
        c = lax.axis_index("c")
        s = lax.axis_index("s")

        @pl.loop(0, CH)
        def _zero(r):
            zv = jnp.zeros((16,), jnp.float32)
            for k in range(DW // 16):
                sbuf[0][r, pl.ds(16 * k, 16)] = zv

        zb = s * RPT
        nfull = RPT // CH
        rem = RPT - nfull * CH
        for j in range(nfull):
            pltpu.sync_copy(sbuf[0], nacc.at[pl.ds(zb + j * CH, CH)])
        pltpu.sync_copy(sbuf[0].at[pl.ds(0, rem)],
                        nacc.at[pl.ds(zb + nfull * CH, rem)])

        def _r16(v):
            u = plsc.bitcast(v, jnp.uint32)
            u = (u + jnp.uint32(0x7FFF) + ((u >> jnp.uint32(16))
                                           & jnp.uint32(1))) & jnp.uint32(0xFFFF0000)
            return plsc.bitcast(u, jnp.float32)

        pltpu.sync_copy(a_h, abuf)
        a_vregs = [_r16(abuf[pl.ds(16 * k, 16)]) for k in range(DD // 16)]

        def issue_idx(k, q):
            base = c * EHALF + (s * CPT + k) * CH
            pltpu.async_copy(src_h.at[pl.ds(base, CH)], sidx[q], semi[q])
            pltpu.async_copy(dst_h.at[pl.ds(base, CH)], didx[q], semi[q])

        def wait_idx(q):
            pltpu.make_async_copy(src_h.at[pl.ds(0, CH)], sidx[q],
                                  semi[q]).wait()
            pltpu.make_async_copy(dst_h.at[pl.ds(0, CH)], didx[q],
                                  semi[q]).wait()

        def issue_gather(i, q):
            pltpu.async_copy(hs_h.at[sidx[q]], sbuf[i], semg[i])
            pltpu.async_copy(ht_h.at[didx[q]], tbuf[i], semg[i])

        def wait_gather(i, q):
            pltpu.make_async_copy(hs_h.at[sidx[q]], sbuf[i], semg[i]).wait()
            pltpu.make_async_copy(ht_h.at[didx[q]], tbuf[i], semg[i]).wait()

        def issue_scatter(i, q):
            pltpu.async_copy(sbuf[i], nacc.at[didx[q]], semsc[i], add=True)

        def wait_scat(i, q):
            pltpu.make_async_copy(sbuf[i], nacc.at[didx[q]], semsc[i]).wait()

        def compute(i):
            sb = sbuf[i]
            tb = tbuf[i]

            @plsc.parallel_loop(0, CH, 1, unroll=4)
            def _edge(b):
                svs = [sb[b, pl.ds(16 * k, 16)] for k in range(DD // 16)]
                tvs = [tb[b, pl.ds(16 * k, 16)] for k in range(DD // 16)]
                accs = [jnp.zeros((16,), jnp.float32) for _ in range(4)]
                for k in range(DD // 16):
                    z = svs[k] + tvs[k]
                    l = _r16(jnp.where(z >= 0, z, 0.2 * z))
                    accs[k % 4] = accs[k % 4] + l * a_vregs[k]
                e = jnp.sum((accs[0] + accs[1]) + (accs[2] + accs[3]))
                ex = jnp.exp(jnp.full((16,), e, jnp.float32))
                for k in range(DD // 16):
                    sb[b, pl.ds(16 * k, 16)] = svs[k] * ex
                sb[b, pl.ds(DD, 16)] = ex

        issue_idx(0, 0)
        wait_idx(0)
        issue_gather(0, 0)
        issue_idx(1, 1)

        plsc.subcore_barrier()

        @pl.loop(0, CPT // 6)
        def _six(kk):
            not_last = kk < CPT // 6 - 1
            for ph in range(6):
                i = ph % 2
                j = 1 - i
                q = ph % 3
                qn = (ph + 1) % 3
                qp = (ph + 2) % 3

                if ph == 0:
                    @pl.when(kk > 0)
                    def _ws():
                        wait_scat(j, qp)
                else:
                    wait_scat(j, qp)

                if ph < 5:
                    wait_idx(qn)
                    issue_gather(j, qn)
                else:
                    @pl.when(not_last)
                    def _pre():
                        wait_idx(qn)
                        issue_gather(j, qn)

                wait_gather(i, q)
                compute(i)
                issue_scatter(i, q)

                if ph < 4:
                    issue_idx(6 * kk + 2 + ph, qp)
                else:
                    @pl.when(not_last)
                    def _nidx():
                        issue_idx(6 * kk + 2 + ph, qp)

        wait_scat((CPT - 1) % 2, (CPT - 1) % 3)

        plsc.subcore_barrier()

        ob = s * RPT
        pltpu.sync_copy(nacc.at[pl.ds(ob, RPT)], out_h.at[c, pl.ds(ob, RPT)])

    return edge_kernel(hs, ht, srcp, dstp, avec)


def _split_num_den(a):
    m = a[0] + a[1]
    num = m[:, :DD]
    den = jnp.max(m[:, DD:DW], axis=-1, keepdims=True)
    return num, den


def _merge_elu_mm2(acc, wa, wb, interpret=False):
    blk = 1000
    grid = NN // blk

    def body(a_ref, wa_ref, wb_ref, oa_ref, ob_ref):
        num, den = _split_num_den(a_ref[...])
        h = num / (den + 1e-16)
        h = jnp.where(h > 0, h, jnp.exp(h) - 1.0)
        ha = _dot16(h, wa_ref[...])
        oa_ref[...] = jnp.concatenate(
            [ha, jnp.ones((blk, 16), jnp.float32)], axis=1)
        ob_ref[...] = _dot16(h, wb_ref[...])

    return pl.pallas_call(
        body,
        grid=(grid,),
        in_specs=[
            pl.BlockSpec((NC, blk, DW), lambda i: (0, i, 0)),
            pl.BlockSpec((DD, DD), lambda i: (0, 0)),
            pl.BlockSpec((DD, DD), lambda i: (0, 0)),
        ],
        out_specs=[
            pl.BlockSpec((blk, DW), lambda i: (i, 0)),
            pl.BlockSpec((blk, DD), lambda i: (i, 0)),
        ],
        out_shape=[
            jax.ShapeDtypeStruct((NN, DW), jnp.float32),
            jax.ShapeDtypeStruct((NN, DD), jnp.float32),
        ],
        interpret=interpret,
    )(acc, wa, wb)


def _heads(acc, batch3, A1, b1, A2, b2, C1, c1, C2, c2, interpret=False):
    blk = 1000
    grid = NN // blk

    def body(a_ref, bt_ref, A1_ref, b1_ref, A2_ref, b2_ref,
             C1_ref, c1_ref, C2_ref, c2_ref, lg_ref, vl_ref, sums, counts):
        i = pl.program_id(0)
        num, den = _split_num_den(a_ref[...])
        emb = num / (den + 1e-16)

        act = jax.nn.gelu(_dot16(emb, A1_ref[...]) + b1_ref[...])
        lg_ref[...] = _dot16(act, A2_ref[...]) + b2_ref[...]

        bb = bt_ref[0]
        oh = (lax.broadcasted_iota(jnp.int32, (NG, blk), 0) == bb).astype(jnp.float32)

        @pl.when(i == 0)
        def _init():
            sums[...] = jnp.zeros((NG, DD), jnp.float32)
            counts[...] = jnp.zeros((NG, 16), jnp.float32)

        sums[...] += jnp.dot(oh, emb, preferred_element_type=jnp.float32,
                             precision=_PREC)
        counts[...] += jnp.broadcast_to(
            jnp.sum(oh, axis=1, keepdims=True), (NG, 16))

        @pl.when(i == grid - 1)
        def _final():
            cnt = jnp.max(counts[...], axis=-1, keepdims=True)
            ge = sums[...] / jnp.maximum(cnt, 1.0)
            ch = jax.nn.gelu(_dot16(ge, C1_ref[...]) + c1_ref[...])
            vl_ref[...] = _dot16(ch, C2_ref[...]) + c2_ref[...]

    return pl.pallas_call(
        body,
        grid=(grid,),
        in_specs=[
            pl.BlockSpec((NC, blk, DW), lambda i: (0, i, 0)),
            pl.BlockSpec((1, 1, blk), lambda i: (i, 0, 0)),
            pl.BlockSpec((DD, DD), lambda i: (0, 0)),
            pl.BlockSpec((1, DD), lambda i: (0, 0)),
            pl.BlockSpec((DD, 1), lambda i: (0, 0)),
            pl.BlockSpec((1, 1), lambda i: (0, 0)),
            pl.BlockSpec((DD, DD), lambda i: (0, 0)),
            pl.BlockSpec((1, DD), lambda i: (0, 0)),
            pl.BlockSpec((DD, 1), lambda i: (0, 0)),
            pl.BlockSpec((1, 1), lambda i: (0, 0)),
        ],
        out_specs=[
            pl.BlockSpec((blk, 1), lambda i: (i, 0)),
            pl.BlockSpec((NG, 1), lambda i: (0, 0)),
        ],
        out_shape=[
            jax.ShapeDtypeStruct((NN, 1), jnp.float32),
            jax.ShapeDtypeStruct((NG, 1), jnp.float32),
        ],
        scratch_shapes=[
            pltpu.VMEM((NG, DD), jnp.float32),
            pltpu.VMEM((NG, 16), jnp.float32),
        ],
        interpret=interpret,
    )(acc, batch3, A1, b1, A2, b2, C1, c1, C2, c2)


def kernel(x, edge_index, batch, W_s1, W_t1, a1, W_s2, W_t2, a2,
           A1, b1, A2, b2, C1, c1, C2, c2):
    src = edge_index[0].astype(jnp.int32)
    dst = edge_index[1].astype(jnp.int32)
    pad = EPAD - EE
    srcp = jnp.concatenate([src, jnp.zeros((pad,), jnp.int32)])
    dstp = jnp.concatenate(
        [dst, NN + (jnp.arange(pad, dtype=jnp.int32) % 16)])
    batch3 = batch.astype(jnp.int32).reshape(NN // 1000, 1, 1000)
    zs = jnp.zeros((NPAD - NN, DW), jnp.float32)
    zt = jnp.zeros((NPAD - NN, DD), jnp.float32)

    hs1, ht1 = _mm2(x, W_s1, W_t1)
    acc1 = _edge_pass(jnp.concatenate([hs1, zs]),
                      jnp.concatenate([ht1, zt]), srcp, dstp, a1)

    hs2, ht2 = _merge_elu_mm2(acc1, W_s2, W_t2)
    acc2 = _edge_pass(jnp.concatenate([hs2, zs]),
                      jnp.concatenate([ht2, zt]), srcp, dstp, a2)

    logits, values = _heads(
        acc2, batch3,
        A1, b1.reshape(1, DD), A2, b2.reshape(1, 1),
        C1, c1.reshape(1, DD), C2, c2.reshape(1, 1))
    return logits.reshape(NN), values


# speedup vs baseline: 1.1320x; 1.1320x over previous
import functools

import jax
import jax.numpy as jnp
from jax import lax
from jax.experimental import pallas as pl
from jax.experimental.pallas import tpu as pltpu
from jax.experimental.pallas import tpu_sc as plsc

NN = 10000
EE = 320000
DD = 128
DW = DD + 16
NG = 16

NC = 2
NS = 16
CH = 56
CPT = 180
EPAD = NC * NS * CPT * CH
EHALF = EPAD // 2
NPAD = NN + 112
RPT = NPAD // NS

_PREC = lax.Precision.HIGHEST


def _dot16(a, b):
    return jnp.dot(a.astype(jnp.bfloat16), b.astype(jnp.bfloat16),
                   preferred_element_type=jnp.float32)


def _mm2(x, wa, wb, interpret=False):
    n = x.shape[0]
    blk = 1000
    grid = n // blk

    def body(x_ref, wa_ref, wb_ref, oa_ref, ob_ref):
        xb = x_ref[...]
        ha = _dot16(xb, wa_ref[...])
        oa_ref[...] = jnp.concatenate(
            [ha, jnp.ones((blk, 16), jnp.float32)], axis=1)
        ob_ref[...] = _dot16(xb, wb_ref[...])

    return pl.pallas_call(
        body,
        grid=(grid,),
        in_specs=[
            pl.BlockSpec((blk, DD), lambda i: (i, 0)),
            pl.BlockSpec((DD, DD), lambda i: (0, 0)),
            pl.BlockSpec((DD, DD), lambda i: (0, 0)),
        ],
        out_specs=[
            pl.BlockSpec((blk, DW), lambda i: (i, 0)),
            pl.BlockSpec((blk, DD), lambda i: (i, 0)),
        ],
        out_shape=[
            jax.ShapeDtypeStruct((n, DW), jnp.float32),
            jax.ShapeDtypeStruct((n, DD), jnp.float32),
        ],
        interpret=interpret,
    )(x, wa, wb)


def _edge_pass(hs, ht, srcp, dstp, avec, interpret=False):
    mesh = plsc.VectorSubcoreMesh(core_axis_name="c", subcore_axis_name="s",
                                  num_cores=NC, num_subcores=NS)

    @functools.partial(
        pl.kernel,
        out_type=jax.ShapeDtypeStruct((NC, NPAD, DW), jnp.float32),
        mesh=mesh,
        scratch_types=[
            pltpu.VMEM_SHARED((NPAD, DW), jnp.float32),
            [pltpu.VMEM((CH,), jnp.int32) for _ in range(3)],
            [pltpu.VMEM((CH,), jnp.int32) for _ in range(3)],
            [pltpu.VMEM((CH, DW), jnp.float32) for _ in range(2)],
            [pltpu.VMEM((CH, DD), jnp.float32) for _ in range(2)],
            pltpu.VMEM((DD,), jnp.float32),
            [pltpu.SemaphoreType.DMA for _ in range(3)],
            [pltpu.SemaphoreType.DMA for _ in range(2)],
            [pltpu.SemaphoreType.DMA for _ in range(2)],
        ],
        compiler_params=pltpu.CompilerParams(needs_layout_passes=False,
                                             use_tc_tiling_on_sc=False),
        interpret=interpret,
    )
    def edge_kernel(hs_h, ht_h, src_h, dst_h, a_h, out_h,
                    nacc, sidx, didx, sbuf, tbuf, abuf, semi, semg, semsc):
        c = lax.axis_index("c")
        s = lax.axis_index("s")

        @pl.loop(0, CH)
        def _zero(r):
            zv = jnp.zeros((16,), jnp.float32)
            for k in range(DW // 16):
                sbuf[0][r, pl.ds(16 * k, 16)] = zv

        zb = s * RPT
        nfull = RPT // CH
        rem = RPT - nfull * CH
        for j in range(nfull):
            pltpu.sync_copy(sbuf[0], nacc.at[pl.ds(zb + j * CH, CH)])
        pltpu.sync_copy(sbuf[0].at[pl.ds(0, rem)],
                        nacc.at[pl.ds(zb + nfull * CH, rem)])

        def _r16(v):
            u = plsc.bitcast(v, jnp.uint32)
            u = (u + jnp.uint32(0x7FFF) + ((u >> jnp.uint32(16))
                                           & jnp.uint32(1))) & jnp.uint32(0xFFFF0000)
            return plsc.bitcast(u, jnp.float32)

        pltpu.sync_copy(a_h, abuf)
        a_vregs = [_r16(abuf[pl.ds(16 * k, 16)]) for k in range(DD // 16)]

        def issue_idx(k, q):
            base = c * EHALF + (s * CPT + k) * CH
            pltpu.async_copy(src_h.at[pl.ds(base, CH)], sidx[q], semi[q])
            pltpu.async_copy(dst_h.at[pl.ds(base, CH)], didx[q], semi[q])

        def wait_idx(q):
            pltpu.make_async_copy(src_h.at[pl.ds(0, CH)], sidx[q],
                                  semi[q]).wait()
            pltpu.make_async_copy(dst_h.at[pl.ds(0, CH)], didx[q],
                                  semi[q]).wait()

        def issue_gather(i, q):
            pltpu.async_copy(hs_h.at[sidx[q]], sbuf[i], semg[i])
            pltpu.async_copy(ht_h.at[didx[q]], tbuf[i], semg[i])

        def wait_gather(i, q):
            pltpu.make_async_copy(hs_h.at[sidx[q]], sbuf[i], semg[i]).wait()
            pltpu.make_async_copy(ht_h.at[didx[q]], tbuf[i], semg[i]).wait()

        def issue_scatter(i, q):
            pltpu.async_copy(sbuf[i], nacc.at[didx[q]], semsc[i], add=True)

        def wait_scat(i, q):
            pltpu.make_async_copy(sbuf[i], nacc.at[didx[q]], semsc[i]).wait()

        def compute(i):
            sb = sbuf[i]
            tb = tbuf[i]

            @plsc.parallel_loop(0, CH, 1, unroll=1)
            def _edge(b):
                svs = [sb[b, pl.ds(16 * k, 16)] for k in range(DD // 16)]
                tvs = [tb[b, pl.ds(16 * k, 16)] for k in range(DD // 16)]
                accs = [jnp.zeros((16,), jnp.float32) for _ in range(4)]
                for k in range(DD // 16):
                    z = svs[k] + tvs[k]
                    l = _r16(jnp.where(z >= 0, z, 0.2 * z))
                    accs[k % 4] = accs[k % 4] + l * a_vregs[k]
                e = jnp.sum((accs[0] + accs[1]) + (accs[2] + accs[3]))
                ex = jnp.exp(jnp.full((16,), e, jnp.float32))
                for k in range(DD // 16):
                    sb[b, pl.ds(16 * k, 16)] = svs[k] * ex
                sb[b, pl.ds(DD, 16)] = ex

        issue_idx(0, 0)
        wait_idx(0)
        issue_gather(0, 0)
        issue_idx(1, 1)

        plsc.subcore_barrier()

        @pl.loop(0, CPT // 6)
        def _six(kk):
            not_last = kk < CPT // 6 - 1
            for ph in range(6):
                i = ph % 2
                j = 1 - i
                q = ph % 3
                qn = (ph + 1) % 3
                qp = (ph + 2) % 3

                if ph == 0:
                    @pl.when(kk > 0)
                    def _ws():
                        wait_scat(j, qp)
                else:
                    wait_scat(j, qp)

                if ph < 5:
                    wait_idx(qn)
                    issue_gather(j, qn)
                else:
                    @pl.when(not_last)
                    def _pre():
                        wait_idx(qn)
                        issue_gather(j, qn)

                wait_gather(i, q)
                compute(i)
                issue_scatter(i, q)

                if ph < 4:
                    issue_idx(6 * kk + 2 + ph, qp)
                else:
                    @pl.when(not_last)
                    def _nidx():
                        issue_idx(6 * kk + 2 + ph, qp)

        wait_scat((CPT - 1) % 2, (CPT - 1) % 3)

        plsc.subcore_barrier()

        ob = s * RPT
        pltpu.sync_copy(nacc.at[pl.ds(ob, RPT)], out_h.at[c, pl.ds(ob, RPT)])

    return edge_kernel(hs, ht, srcp, dstp, avec)


def _split_num_den(a):
    m = a[0] + a[1]
    num = m[:, :DD]
    den = jnp.max(m[:, DD:DW], axis=-1, keepdims=True)
    return num, den


def _merge_elu_mm2(acc, wa, wb, interpret=False):
    blk = 1000
    grid = NN // blk

    def body(a_ref, wa_ref, wb_ref, oa_ref, ob_ref):
        num, den = _split_num_den(a_ref[...])
        h = num / (den + 1e-16)
        h = jnp.where(h > 0, h, jnp.exp(h) - 1.0)
        ha = _dot16(h, wa_ref[...])
        oa_ref[...] = jnp.concatenate(
            [ha, jnp.ones((blk, 16), jnp.float32)], axis=1)
        ob_ref[...] = _dot16(h, wb_ref[...])

    return pl.pallas_call(
        body,
        grid=(grid,),
        in_specs=[
            pl.BlockSpec((NC, blk, DW), lambda i: (0, i, 0)),
            pl.BlockSpec((DD, DD), lambda i: (0, 0)),
            pl.BlockSpec((DD, DD), lambda i: (0, 0)),
        ],
        out_specs=[
            pl.BlockSpec((blk, DW), lambda i: (i, 0)),
            pl.BlockSpec((blk, DD), lambda i: (i, 0)),
        ],
        out_shape=[
            jax.ShapeDtypeStruct((NN, DW), jnp.float32),
            jax.ShapeDtypeStruct((NN, DD), jnp.float32),
        ],
        interpret=interpret,
    )(acc, wa, wb)


def _heads(acc, batch3, A1, b1, A2, b2, C1, c1, C2, c2, interpret=False):
    blk = 1000
    grid = NN // blk

    def body(a_ref, bt_ref, A1_ref, b1_ref, A2_ref, b2_ref,
             C1_ref, c1_ref, C2_ref, c2_ref, lg_ref, vl_ref, sums, counts):
        i = pl.program_id(0)
        num, den = _split_num_den(a_ref[...])
        emb = num / (den + 1e-16)

        act = jax.nn.gelu(_dot16(emb, A1_ref[...]) + b1_ref[...])
        lg_ref[...] = _dot16(act, A2_ref[...]) + b2_ref[...]

        bb = bt_ref[0]
        oh = (lax.broadcasted_iota(jnp.int32, (NG, blk), 0) == bb).astype(jnp.float32)

        @pl.when(i == 0)
        def _init():
            sums[...] = jnp.zeros((NG, DD), jnp.float32)
            counts[...] = jnp.zeros((NG, 16), jnp.float32)

        sums[...] += jnp.dot(oh, emb, preferred_element_type=jnp.float32,
                             precision=_PREC)
        counts[...] += jnp.broadcast_to(
            jnp.sum(oh, axis=1, keepdims=True), (NG, 16))

        @pl.when(i == grid - 1)
        def _final():
            cnt = jnp.max(counts[...], axis=-1, keepdims=True)
            ge = sums[...] / jnp.maximum(cnt, 1.0)
            ch = jax.nn.gelu(_dot16(ge, C1_ref[...]) + c1_ref[...])
            vl_ref[...] = _dot16(ch, C2_ref[...]) + c2_ref[...]

    return pl.pallas_call(
        body,
        grid=(grid,),
        in_specs=[
            pl.BlockSpec((NC, blk, DW), lambda i: (0, i, 0)),
            pl.BlockSpec((1, 1, blk), lambda i: (i, 0, 0)),
            pl.BlockSpec((DD, DD), lambda i: (0, 0)),
            pl.BlockSpec((1, DD), lambda i: (0, 0)),
            pl.BlockSpec((DD, 1), lambda i: (0, 0)),
            pl.BlockSpec((1, 1), lambda i: (0, 0)),
            pl.BlockSpec((DD, DD), lambda i: (0, 0)),
            pl.BlockSpec((1, DD), lambda i: (0, 0)),
            pl.BlockSpec((DD, 1), lambda i: (0, 0)),
            pl.BlockSpec((1, 1), lambda i: (0, 0)),
        ],
        out_specs=[
            pl.BlockSpec((blk, 1), lambda i: (i, 0)),
            pl.BlockSpec((NG, 1), lambda i: (0, 0)),
        ],
        out_shape=[
            jax.ShapeDtypeStruct((NN, 1), jnp.float32),
            jax.ShapeDtypeStruct((NG, 1), jnp.float32),
        ],
        scratch_shapes=[
            pltpu.VMEM((NG, DD), jnp.float32),
            pltpu.VMEM((NG, 16), jnp.float32),
        ],
        interpret=interpret,
    )(acc, batch3, A1, b1, A2, b2, C1, c1, C2, c2)


def kernel(x, edge_index, batch, W_s1, W_t1, a1, W_s2, W_t2, a2,
           A1, b1, A2, b2, C1, c1, C2, c2):
    src = edge_index[0].astype(jnp.int32)
    dst = edge_index[1].astype(jnp.int32)
    pad = EPAD - EE
    srcp = jnp.concatenate([src, jnp.zeros((pad,), jnp.int32)])
    dstp = jnp.concatenate(
        [dst, NN + (jnp.arange(pad, dtype=jnp.int32) % 16)])
    batch3 = batch.astype(jnp.int32).reshape(NN // 1000, 1, 1000)
    zs = jnp.zeros((NPAD - NN, DW), jnp.float32)
    zt = jnp.zeros((NPAD - NN, DD), jnp.float32)

    hs1, ht1 = _mm2(x, W_s1, W_t1)
    acc1 = _edge_pass(jnp.concatenate([hs1, zs]),
                      jnp.concatenate([ht1, zt]), srcp, dstp, a1)

    hs2, ht2 = _merge_elu_mm2(acc1, W_s2, W_t2)
    acc2 = _edge_pass(jnp.concatenate([hs2, zs]),
                      jnp.concatenate([ht2, zt]), srcp, dstp, a2)

    logits, values = _heads(
        acc2, batch3,
        A1, b1.reshape(1, DD), A2, b2.reshape(1, 1),
        C1, c1.reshape(1, DD), C2, c2.reshape(1, 1))
    return logits.reshape(NN), values
